# baseline (speedup 1.0000x reference)
import jax
import jax.numpy as jnp
from jax import lax
from jax.experimental import pallas as pl
from jax.experimental.pallas import tpu as pltpu
from jax.experimental.pallas import tpu_sc as plsc

_NC = 2
_NS = 16
_NW = _NC * _NS
_C = 16
_K = 200
_PER_TILE = _C * _K
_NPAD = _NW * _PER_TILE
_H = 256
_W = _H // 2
_F = 9
_ROWS = 900


def _sc_body(gidx_hbm, tab_hbm, out_hbm,
             table_v, idx_a, idx_b, out_a, out_b,
             sem_ia, sem_ib, sem_oa, sem_ob):
    c = lax.axis_index("c")
    s = lax.axis_index("s")
    wid = s * _NC + c

    pltpu.sync_copy(tab_hbm, table_v)

    lanes = lax.iota(jnp.int32, 16)
    k2 = _K // 2

    def compute_chunk(idx_v, out_v):
        base = [idx_v[i] for i in range(_F)]

        @plsc.parallel_loop(0, _W, unroll=8)
        def col_body(col):
            colperm = (lanes + col) & (_W - 1)
            words = [
                plsc.load_gather(table_v, [base[i] + colperm])
                for i in range(_F)
            ]
            los = [plsc.bitcast(w << 16, jnp.float32) for w in words]
            his = [plsc.bitcast(w, jnp.float32) for w in words]

            def tree(vals):
                while len(vals) > 1:
                    vals = [
                        vals[t] + vals[t + 1]
                        if t + 1 < len(vals) else vals[t]
                        for t in range(0, len(vals), 2)
                    ]
                return vals[0]

            two_col = colperm << 1
            plsc.store_scatter(out_v, [lanes, two_col], tree(los))
            plsc.store_scatter(out_v, [lanes, two_col + 1], tree(his))

    def out_slice(k):
        return out_hbm.at[pl.ds(wid * _PER_TILE + k * _C, _C)]

    pltpu.async_copy(gidx_hbm.at[wid, 0], idx_a, sem_ia)

    def pair_body(kk, carry):
        k0 = kk * 2
        k1 = k0 + 1
        cp_b = pltpu.async_copy(gidx_hbm.at[wid, k1], idx_b, sem_ib)
        pltpu.make_async_copy(gidx_hbm.at[wid, k0], idx_a, sem_ia).wait()

        @pl.when(kk > 0)
        def _():
            pltpu.make_async_copy(out_a, out_slice(k0), sem_oa).wait()

        compute_chunk(idx_a, out_a)
        pltpu.async_copy(out_a, out_slice(k0), sem_oa)

        @pl.when(kk + 1 < k2)
        def _():
            pltpu.async_copy(gidx_hbm.at[wid, k0 + 2], idx_a, sem_ia)

        cp_b.wait()

        @pl.when(kk > 0)
        def _():
            pltpu.make_async_copy(out_b, out_slice(k1), sem_ob).wait()

        compute_chunk(idx_b, out_b)
        pltpu.async_copy(out_b, out_slice(k1), sem_ob)
        return carry

    lax.fori_loop(0, k2, pair_body, 0, unroll=False)

    pltpu.make_async_copy(out_a, out_slice(0), sem_oa).wait()
    pltpu.make_async_copy(out_b, out_slice(1), sem_ob).wait()


def kernel(x, emb):
    n, f = x.shape
    _, v, h = emb.shape
    gidx = (x + v * jnp.arange(f, dtype=jnp.int32)[None, :]) * _W
    gidx = jnp.zeros((_NPAD, f), jnp.int32).at[:n].set(gidx)
    gidx4 = gidx.reshape(_NW, _K, _C, f).transpose(0, 1, 3, 2)
    emb_flat = emb.reshape(f * v, h)
    u16 = lax.bitcast_convert_type(
        emb_flat.astype(jnp.bfloat16), jnp.uint16
    ).astype(jnp.uint32)
    packed = (u16[:, 0::2] | (u16[:, 1::2] << 16)).astype(jnp.int32)
    packed = packed.reshape(-1)

    mesh = plsc.VectorSubcoreMesh(
        core_axis_name="c", subcore_axis_name="s",
        num_cores=_NC, num_subcores=_NS,
    )
    run = pl.kernel(
        _sc_body,
        out_type=jax.ShapeDtypeStruct((_NPAD, h), jnp.float32),
        mesh=mesh,
        scratch_types=[
            pltpu.VMEM((_ROWS * _W,), jnp.int32),
            pltpu.VMEM((_F, _C), jnp.int32),
            pltpu.VMEM((_F, _C), jnp.int32),
            pltpu.VMEM((_C, _H), jnp.float32),
            pltpu.VMEM((_C, _H), jnp.float32),
            pltpu.SemaphoreType.DMA,
            pltpu.SemaphoreType.DMA,
            pltpu.SemaphoreType.DMA,
            pltpu.SemaphoreType.DMA,
        ],
        compiler_params=pltpu.CompilerParams(needs_layout_passes=False),
    )
    out = run(gidx4, packed)
    return out[:n]


# speedup vs baseline: 1.0039x; 1.0039x over previous
import jax
import jax.numpy as jnp
from jax import lax
from jax.experimental import pallas as pl
from jax.experimental.pallas import tpu as pltpu
from jax.experimental.pallas import tpu_sc as plsc

_NC = 2
_NS = 16
_NW = _NC * _NS
_C = 16
_K = 200
_PER_TILE = _C * _K
_NPAD = _NW * _PER_TILE
_H = 256
_W = _H // 2
_F = 9
_ROWS = 900


def _sc_body(gidx_hbm, tab_hbm, out_hbm,
             table_v, idx_a, idx_b, out_a, out_b,
             sem_ia, sem_ib, sem_oa, sem_ob):
    c = lax.axis_index("c")
    s = lax.axis_index("s")
    wid = s * _NC + c

    pltpu.sync_copy(tab_hbm, table_v)

    lanes = lax.iota(jnp.int32, 16)
    k2 = _K // 2

    def compute_chunk(idx_v, out_v):
        base = [idx_v[i] for i in range(_F)]

        @plsc.parallel_loop(0, _W, unroll=16)
        def col_body(col):
            colperm = (lanes + col) & (_W - 1)
            words = [
                plsc.load_gather(table_v, [base[i] + colperm])
                for i in range(_F)
            ]
            los = [plsc.bitcast(w << 16, jnp.float32) for w in words]
            his = [plsc.bitcast(w, jnp.float32) for w in words]

            def tree(vals):
                while len(vals) > 1:
                    vals = [
                        vals[t] + vals[t + 1]
                        if t + 1 < len(vals) else vals[t]
                        for t in range(0, len(vals), 2)
                    ]
                return vals[0]

            two_col = colperm << 1
            plsc.store_scatter(out_v, [lanes, two_col], tree(los))
            plsc.store_scatter(out_v, [lanes, two_col + 1], tree(his))

    def out_slice(k):
        return out_hbm.at[pl.ds(wid * _PER_TILE + k * _C, _C)]

    pltpu.async_copy(gidx_hbm.at[wid, 0], idx_a, sem_ia)

    def pair_body(kk, carry):
        k0 = kk * 2
        k1 = k0 + 1
        cp_b = pltpu.async_copy(gidx_hbm.at[wid, k1], idx_b, sem_ib)
        pltpu.make_async_copy(gidx_hbm.at[wid, k0], idx_a, sem_ia).wait()

        @pl.when(kk > 0)
        def _():
            pltpu.make_async_copy(out_a, out_slice(k0), sem_oa).wait()

        compute_chunk(idx_a, out_a)
        pltpu.async_copy(out_a, out_slice(k0), sem_oa)

        @pl.when(kk + 1 < k2)
        def _():
            pltpu.async_copy(gidx_hbm.at[wid, k0 + 2], idx_a, sem_ia)

        cp_b.wait()

        @pl.when(kk > 0)
        def _():
            pltpu.make_async_copy(out_b, out_slice(k1), sem_ob).wait()

        compute_chunk(idx_b, out_b)
        pltpu.async_copy(out_b, out_slice(k1), sem_ob)
        return carry

    lax.fori_loop(0, k2, pair_body, 0, unroll=False)

    pltpu.make_async_copy(out_a, out_slice(0), sem_oa).wait()
    pltpu.make_async_copy(out_b, out_slice(1), sem_ob).wait()


def kernel(x, emb):
    n, f = x.shape
    _, v, h = emb.shape
    gidx = (x + v * jnp.arange(f, dtype=jnp.int32)[None, :]) * _W
    gidx = jnp.zeros((_NPAD, f), jnp.int32).at[:n].set(gidx)
    gidx4 = gidx.reshape(_NW, _K, _C, f).transpose(0, 1, 3, 2)
    emb_flat = emb.reshape(f * v, h)
    u16 = lax.bitcast_convert_type(
        emb_flat.astype(jnp.bfloat16), jnp.uint16
    ).astype(jnp.uint32)
    packed = (u16[:, 0::2] | (u16[:, 1::2] << 16)).astype(jnp.int32)
    packed = packed.reshape(-1)

    mesh = plsc.VectorSubcoreMesh(
        core_axis_name="c", subcore_axis_name="s",
        num_cores=_NC, num_subcores=_NS,
    )
    run = pl.kernel(
        _sc_body,
        out_type=jax.ShapeDtypeStruct((_NPAD, h), jnp.float32),
        mesh=mesh,
        scratch_types=[
            pltpu.VMEM((_ROWS * _W,), jnp.int32),
            pltpu.VMEM((_F, _C), jnp.int32),
            pltpu.VMEM((_F, _C), jnp.int32),
            pltpu.VMEM((_C, _H), jnp.float32),
            pltpu.VMEM((_C, _H), jnp.float32),
            pltpu.SemaphoreType.DMA,
            pltpu.SemaphoreType.DMA,
            pltpu.SemaphoreType.DMA,
            pltpu.SemaphoreType.DMA,
        ],
        compiler_params=pltpu.CompilerParams(needs_layout_passes=False),
    )
    out = run(gidx4, packed)
    return out[:n]
